# baseline (speedup 1.0000x reference)
import functools

import jax
import jax.numpy as jnp
from jax import lax
from jax.experimental import pallas as pl
from jax.experimental.pallas import tpu as pltpu
from jax.experimental.pallas import tpu_sc as plsc

THETA_MIN = 1.0
THETA_MAX = 5.0
A_MIN = 1.0
A_MAX = 3.0

BATCH = 16384
NC = 2
NS = 16
NW = NC * NS
BPW = BATCH // NW
L = 16


def _sigmoid(x):
    return 1.0 / (1.0 + jnp.exp(-x))


_mesh = plsc.VectorSubcoreMesh(core_axis_name="c", subcore_axis_name="s")


@functools.partial(
    pl.kernel,
    mesh=_mesh,
    out_type=jax.ShapeDtypeStruct((BATCH,), jnp.float32),
    scratch_types=[
        pltpu.VMEM((BPW,), jnp.int32),
        pltpu.VMEM((BPW,), jnp.int32),
        pltpu.VMEM((BPW,), jnp.float32),
        pltpu.VMEM((BPW,), jnp.float32),
        pltpu.VMEM((BPW,), jnp.float32),
        pltpu.VMEM((BPW,), jnp.float32),
        pltpu.SemaphoreType.DMA,
        pltpu.SemaphoreType.DMA,
    ],
)
def _irt_sc_kernel(theta_hbm, a_hbm, b_hbm, rid_hbm, iid_hbm, out_hbm,
                   rid_v, iid_v, th_v, av_v, bv_v, out_v, isem, gsem):
    wid = lax.axis_index("s") * NC + lax.axis_index("c")
    base = wid * BPW

    rcp = pltpu.async_copy(rid_hbm.at[pl.ds(base, BPW)], rid_v, isem)
    icp = pltpu.async_copy(iid_hbm.at[pl.ds(base, BPW)], iid_v, isem)
    rcp.wait()
    icp.wait()

    tcp = pltpu.async_copy(theta_hbm.at[rid_v], th_v, gsem)
    acp = pltpu.async_copy(a_hbm.at[iid_v], av_v, gsem)
    bcp = pltpu.async_copy(b_hbm.at[iid_v], bv_v, gsem)
    tcp.wait()
    acp.wait()
    bcp.wait()

    for i in range(BPW // L):
        s = pl.ds(i * L, L)
        theta = _sigmoid(th_v[s]) * (THETA_MAX - THETA_MIN) + THETA_MIN
        item_a = _sigmoid(av_v[s]) * (A_MAX - A_MIN) + A_MIN
        item_b = _sigmoid(bv_v[s]) * (THETA_MAX - THETA_MIN) + THETA_MIN
        out_v[s] = _sigmoid(item_a * (theta - item_b))

    pltpu.sync_copy(out_v, out_hbm.at[pl.ds(base, BPW)])


def kernel(respondent_ids, item_ids, a_raw, b_raw, theta_raw):
    return _irt_sc_kernel(
        theta_raw.reshape(-1),
        a_raw.reshape(-1),
        b_raw.reshape(-1),
        respondent_ids.astype(jnp.int32),
        item_ids.astype(jnp.int32),
    )


# speedup vs baseline: 1.0331x; 1.0331x over previous
import functools

import jax
import jax.numpy as jnp
from jax import lax
from jax.experimental import pallas as pl
from jax.experimental.pallas import tpu as pltpu
from jax.experimental.pallas import tpu_sc as plsc

THETA_MIN = 1.0
THETA_MAX = 5.0
A_MIN = 1.0
A_MAX = 3.0

BATCH = 16384
NC = 2
NS = 16
NW = NC * NS
BPW = BATCH // NW
L = 16


def _sigmoid(x):
    return 1.0 / (1.0 + jnp.exp(-x))


_mesh = plsc.VectorSubcoreMesh(core_axis_name="c", subcore_axis_name="s")


@functools.partial(
    pl.kernel,
    mesh=_mesh,
    out_type=jax.ShapeDtypeStruct((BATCH,), jnp.float32),
    scratch_types=[
        pltpu.VMEM((BPW,), jnp.int32),
        pltpu.VMEM((BPW,), jnp.int32),
        pltpu.VMEM((BPW,), jnp.float32),
        pltpu.VMEM((BPW,), jnp.float32),
        pltpu.VMEM((BPW,), jnp.float32),
        pltpu.VMEM((BPW,), jnp.float32),
        pltpu.SemaphoreType.DMA,
        pltpu.SemaphoreType.DMA,
    ],
)
def _irt_sc_kernel(theta_hbm, a_hbm, b_hbm, rid_hbm, iid_hbm, out_hbm,
                   rid_v, iid_v, th_v, av_v, bv_v, out_v, isem, gsem):
    wid = lax.axis_index("s") * NC + lax.axis_index("c")
    base = wid * BPW

    rcp = pltpu.async_copy(rid_hbm.at[pl.ds(base, BPW)], rid_v, isem)
    icp = pltpu.async_copy(iid_hbm.at[pl.ds(base, BPW)], iid_v, isem)
    rcp.wait()
    icp.wait()

    tcp = pltpu.async_copy(theta_hbm.at[rid_v], th_v, gsem)
    acp = pltpu.async_copy(a_hbm.at[iid_v], av_v, gsem)
    bcp = pltpu.async_copy(b_hbm.at[iid_v], bv_v, gsem)
    tcp.wait()
    acp.wait()
    bcp.wait()

    def body(i, carry):
        s = pl.ds(i * L, L)
        theta = _sigmoid(th_v[s]) * (THETA_MAX - THETA_MIN) + THETA_MIN
        item_a = _sigmoid(av_v[s]) * (A_MAX - A_MIN) + A_MIN
        item_b = _sigmoid(bv_v[s]) * (THETA_MAX - THETA_MIN) + THETA_MIN
        out_v[s] = _sigmoid(item_a * (theta - item_b))
        return carry

    lax.fori_loop(0, BPW // L, body, 0)

    pltpu.sync_copy(out_v, out_hbm.at[pl.ds(base, BPW)])


def kernel(respondent_ids, item_ids, a_raw, b_raw, theta_raw):
    return _irt_sc_kernel(
        theta_raw.reshape(-1),
        a_raw.reshape(-1),
        b_raw.reshape(-1),
        respondent_ids.astype(jnp.int32),
        item_ids.astype(jnp.int32),
    )


# speedup vs baseline: 1.0656x; 1.0314x over previous
import functools

import jax
import jax.numpy as jnp
from jax import lax
from jax.experimental import pallas as pl
from jax.experimental.pallas import tpu as pltpu
from jax.experimental.pallas import tpu_sc as plsc

THETA_MIN = 1.0
THETA_MAX = 5.0
A_MIN = 1.0
A_MAX = 3.0

BATCH = 16384
NC = 2
NS = 16
NW = NC * NS
BPW = BATCH // NW
L = 16


def _sigmoid(x):
    return 1.0 / (1.0 + jnp.exp(-x))


_mesh = plsc.VectorSubcoreMesh(core_axis_name="c", subcore_axis_name="s")


@functools.partial(
    pl.kernel,
    mesh=_mesh,
    out_type=(
        jax.ShapeDtypeStruct((BATCH,), jnp.float32),
        jax.ShapeDtypeStruct((BATCH,), jnp.float32),
    ),
    scratch_types=[
        pltpu.VMEM((BPW,), jnp.int32),
        pltpu.VMEM((BPW,), jnp.float32),
        pltpu.VMEM((BPW,), jnp.float32),
        pltpu.VMEM((BPW,), jnp.float32),
        pltpu.VMEM((BPW,), jnp.float32),
        pltpu.SemaphoreType.DMA,
        pltpu.SemaphoreType.DMA,
    ],
)
def _item_sc_kernel(a_hbm, b_hbm, iid_hbm, pa_hbm, pb_hbm,
                    iid_v, av_v, bv_v, pa_v, pb_v, isem, gsem):
    wid = lax.axis_index("s") * NC + lax.axis_index("c")
    base = wid * BPW

    pltpu.async_copy(iid_hbm.at[pl.ds(base, BPW)], iid_v, isem).wait()
    acp = pltpu.async_copy(a_hbm.at[iid_v], av_v, gsem)
    bcp = pltpu.async_copy(b_hbm.at[iid_v], bv_v, gsem)
    acp.wait()
    bcp.wait()

    def body(i, carry):
        s = pl.ds(i * L, L)
        pa_v[s] = _sigmoid(av_v[s]) * (A_MAX - A_MIN) + A_MIN
        pb_v[s] = _sigmoid(bv_v[s]) * (THETA_MAX - THETA_MIN) + THETA_MIN
        return carry

    lax.fori_loop(0, BPW // L, body, 0)

    pcp = pltpu.async_copy(pa_v, pa_hbm.at[pl.ds(base, BPW)], isem)
    qcp = pltpu.async_copy(pb_v, pb_hbm.at[pl.ds(base, BPW)], isem)
    pcp.wait()
    qcp.wait()


@functools.partial(
    pl.kernel,
    mesh=_mesh,
    out_type=jax.ShapeDtypeStruct((BATCH,), jnp.float32),
    scratch_types=[
        pltpu.VMEM((BPW,), jnp.int32),
        pltpu.VMEM((BPW,), jnp.float32),
        pltpu.VMEM((BPW,), jnp.float32),
        pltpu.VMEM((BPW,), jnp.float32),
        pltpu.VMEM((BPW,), jnp.float32),
        pltpu.SemaphoreType.DMA,
        pltpu.SemaphoreType.DMA,
    ],
)
def _combine_sc_kernel(theta_hbm, pa_hbm, pb_hbm, rid_hbm, out_hbm,
                       rid_v, th_v, pa_v, pb_v, out_v, isem, gsem):
    wid = lax.axis_index("s") * NC + lax.axis_index("c")
    base = wid * BPW

    rcp = pltpu.async_copy(rid_hbm.at[pl.ds(base, BPW)], rid_v, isem)
    pcp = pltpu.async_copy(pa_hbm.at[pl.ds(base, BPW)], pa_v, isem)
    qcp = pltpu.async_copy(pb_hbm.at[pl.ds(base, BPW)], pb_v, isem)
    rcp.wait()
    tcp = pltpu.async_copy(theta_hbm.at[rid_v], th_v, gsem)
    pcp.wait()
    qcp.wait()
    tcp.wait()

    def body(i, carry):
        s = pl.ds(i * L, L)
        theta = _sigmoid(th_v[s]) * (THETA_MAX - THETA_MIN) + THETA_MIN
        out_v[s] = _sigmoid(pa_v[s] * (theta - pb_v[s]))
        return carry

    lax.fori_loop(0, BPW // L, body, 0)

    pltpu.sync_copy(out_v, out_hbm.at[pl.ds(base, BPW)])


def kernel(respondent_ids, item_ids, a_raw, b_raw, theta_raw):
    rid = respondent_ids.astype(jnp.int32)
    iid = item_ids.astype(jnp.int32)
    pa, pb = _item_sc_kernel(a_raw.reshape(-1), b_raw.reshape(-1), iid)
    return _combine_sc_kernel(theta_raw.reshape(-1), pa, pb, rid)
